# Initial kernel scaffold; baseline (speedup 1.0000x reference)
import jax
import jax.numpy as jnp
from jax.experimental import pallas as pl


def kernel(y_p, y_t, chain_ba_1, chain_ba_2, chain_da_1, chain_da_2, chain_da_3, sign_1, sign_2, sign_3):
    raise NotImplementedError("write your pallas kernel here")


# speedup vs baseline: 7.3948x; 7.3948x over previous
import functools

import jax
import jax.numpy as jnp
from jax import lax
from jax.experimental import pallas as pl
from jax.experimental.pallas import tpu as pltpu
from jax.experimental.pallas import tpu_sc as plsc

_B = 64
_N = 4096
_N_BA = 4095
_N_DA = 4094
_NW = 32
_KPW = _N // _NW
_L = 16
_NG = _B // _L
_D = 3 * _B


def _rsqrt(x):
    i = plsc.bitcast(x, jnp.int32)
    y = plsc.bitcast(jnp.int32(0x5F3759DF) - (i >> 1), jnp.float32)
    y = y * (1.5 - 0.5 * x * y * y)
    y = y * (1.5 - 0.5 * x * y * y)
    return y


def _cos_ba(ax, ay, az, bx, by, bz):
    inner = -(ax * bx + ay * by + az * bz)
    den = (ax * ax + ay * ay + az * az) * (bx * bx + by * by + bz * bz)
    c = inner * _rsqrt(den)
    return jnp.where(den == 0.0, jnp.zeros_like(c), c)


def _cos_da(ax, ay, az, bx, by, bz, cx, cy, cz):
    c1x = ay * bz - az * by
    c1y = az * bx - ax * bz
    c1z = ax * by - ay * bx
    c2x = by * cz - bz * cy
    c2y = bz * cx - bx * cz
    c2z = bx * cy - by * cx
    inner = c1x * c2x + c1y * c2y + c1z * c2z
    den = (c1x * c1x + c1y * c1y + c1z * c1z) * (c2x * c2x + c2y * c2y + c2z * c2z)
    c = inner * _rsqrt(den)
    return jnp.where(den == 0.0, jnp.zeros_like(c), c)


def _row(ref, k, g):
    o = g * _L
    return (ref[k, pl.ds(o, _L)],
            ref[k, pl.ds(_B + o, _L)],
            ref[k, pl.ds(2 * _B + o, _L)])


_mesh = plsc.VectorSubcoreMesh(core_axis_name="c", subcore_axis_name="s")


@functools.partial(
    pl.kernel,
    mesh=_mesh,
    compiler_params=pltpu.CompilerParams(needs_layout_passes=False,
                                         use_tc_tiling_on_sc=False),
    out_type=jax.ShapeDtypeStruct((_NW, _L), jnp.float32),
    scratch_types=[
        pltpu.VMEM((_KPW, _D), jnp.float32),
        pltpu.VMEM((_KPW, _D), jnp.float32),
        pltpu.VMEM((_KPW, _D), jnp.float32),
        pltpu.VMEM((_KPW, _D), jnp.float32),
        pltpu.VMEM((_KPW // 2, _D), jnp.float32),
        pltpu.VMEM((_KPW // 2, _D), jnp.float32),
        pltpu.VMEM((_KPW,), jnp.int32),
        pltpu.VMEM((_KPW,), jnp.int32),
        pltpu.VMEM((2, _KPW // 2), jnp.int32),
        pltpu.VMEM((_KPW,), jnp.float32),
        pltpu.VMEM((_KPW,), jnp.float32),
        pltpu.VMEM((_KPW,), jnp.float32),
        pltpu.VMEM((_KPW + _L,), jnp.float32),
        pltpu.VMEM((_L,), jnp.float32),
        pltpu.SemaphoreType.DMA,
    ],
)
def _sc_loss(yp_hbm, yt_hbm, ba1_hbm, ba2_hbm, da1_hbm, da2_hbm, da3_hbm,
             s1_hbm, s2_hbm, s3_hbm,
             out_hbm, A1, A2, A3, A4, C1, C2, I1, I2, I3, S1, S2, S3, FW, OB,
             sem):
    cid = lax.axis_index("c")
    sid = lax.axis_index("s")
    wid = sid * 2 + cid
    base = wid * _KPW
    zero = jnp.zeros((_L,), jnp.float32)

    pltpu.sync_copy(yp_hbm.at[pl.ds(base, _KPW)], A1)
    pltpu.sync_copy(yt_hbm.at[pl.ds(base, _KPW)], A2)

    def bl_body(k, acc):
        for g in range(_NG):
            px, py, pz = _row(A1, k, g)
            tx, ty, tz = _row(A2, k, g)
            sp = px * px + py * py + pz * pz
            st = tx * tx + ty * ty + tz * tz
            acc = acc + jnp.abs(st * _rsqrt(st) - sp * _rsqrt(sp))
        return acc

    acc_bl = lax.fori_loop(0, _KPW, bl_body, zero)

    pltpu.sync_copy(ba1_hbm.at[wid], I1)
    pltpu.sync_copy(ba2_hbm.at[wid], I2)
    g1 = pltpu.async_copy(yp_hbm.at[I1], A1, sem)
    g2 = pltpu.async_copy(yt_hbm.at[I1], A2, sem)
    g3 = pltpu.async_copy(yp_hbm.at[I2], A3, sem)
    g4 = pltpu.async_copy(yt_hbm.at[I2], A4, sem)
    g1.wait(); g2.wait(); g3.wait(); g4.wait()

    def ba_body(k, acc):
        kacc = zero
        for g in range(_NG):
            p1 = _row(A1, k, g)
            t1 = _row(A2, k, g)
            p2 = _row(A3, k, g)
            t2 = _row(A4, k, g)
            kacc = kacc + jnp.abs(_cos_ba(*t1, *t2) - _cos_ba(*p1, *p2))
        w = jnp.where(base + k < _N_BA, 1.0, 0.0).astype(jnp.float32)
        return acc + kacc * w

    acc_ba = lax.fori_loop(0, _KPW, ba_body, zero)

    pltpu.sync_copy(da1_hbm.at[wid], I1)
    pltpu.sync_copy(da2_hbm.at[wid], I2)
    pltpu.sync_copy(da3_hbm.at[wid], I3)
    pltpu.sync_copy(s1_hbm.at[wid], S1)
    pltpu.sync_copy(s2_hbm.at[wid], S2)
    pltpu.sync_copy(s3_hbm.at[wid], S3)
    g1 = pltpu.async_copy(yp_hbm.at[I1], A1, sem)
    g2 = pltpu.async_copy(yt_hbm.at[I1], A2, sem)
    g3 = pltpu.async_copy(yp_hbm.at[I2], A3, sem)
    g4 = pltpu.async_copy(yt_hbm.at[I2], A4, sem)
    g1.wait(); g2.wait(); g3.wait(); g4.wait()

    for c in range(_KPW // _L):
        o = c * _L
        sa = S1[pl.ds(o, _L)]
        sb = S2[pl.ds(o, _L)]
        sc = S3[pl.ds(o, _L)]
        num = jnp.abs(sa * sb * sb * sc)
        den = jnp.abs(sa * sb) * jnp.abs(sb * sc)
        safe = jnp.where(den == 0.0, jnp.ones_like(den), den)
        FW[pl.ds(o, _L)] = jnp.where(den == 0.0, jnp.zeros_like(num), num / safe)
    FW[pl.ds(_KPW, _L)] = zero

    acc_da = zero
    for h in range(2):
        g5 = pltpu.async_copy(yp_hbm.at[I3.at[h]], C1, sem)
        g6 = pltpu.async_copy(yt_hbm.at[I3.at[h]], C2, sem)
        g5.wait(); g6.wait()

        def da_body(k, acc, h=h):
            kk = h * (_KPW // 2) + k
            kacc = zero
            for g in range(_NG):
                p1 = _row(A1, kk, g)
                t1 = _row(A2, kk, g)
                p2 = _row(A3, kk, g)
                t2 = _row(A4, kk, g)
                p3 = _row(C1, k, g)
                t3 = _row(C2, k, g)
                kacc = kacc + jnp.abs(_cos_da(*t1, *t2, *t3) - _cos_da(*p1, *p2, *p3))
            fw = FW[pl.ds(kk, _L)][0]
            w = jnp.where(base + kk < _N_DA, fw, 0.0).astype(jnp.float32)
            return acc + kacc * w

        acc_da = lax.fori_loop(0, _KPW // 2, da_body, acc_da)

    partial = (acc_bl * (1.0 / (_B * _N))
               + acc_ba * (1.0 / (_B * _N_BA))
               + acc_da * (1.0 / (_B * _N_DA)))
    OB[...] = partial
    pltpu.sync_copy(OB, out_hbm.at[wid])


def _pad_i32(a, n):
    return jnp.concatenate([a.astype(jnp.int32),
                            jnp.zeros((n - a.shape[0],), jnp.int32)])


def _pad_f32(a, n):
    return jnp.concatenate([a.astype(jnp.float32),
                            jnp.ones((n - a.shape[0],), jnp.float32)])


def kernel(y_p, y_t, chain_ba_1, chain_ba_2, chain_da_1, chain_da_2,
           chain_da_3, sign_1, sign_2, sign_3):
    yp2 = y_p.transpose(1, 2, 0).reshape(_N, _D)
    yt2 = y_t.transpose(1, 2, 0).reshape(_N, _D)
    ba1 = _pad_i32(chain_ba_1, _N).reshape(_NW, _KPW)
    ba2 = _pad_i32(chain_ba_2, _N).reshape(_NW, _KPW)
    da1 = _pad_i32(chain_da_1, _N).reshape(_NW, _KPW)
    da2 = _pad_i32(chain_da_2, _N).reshape(_NW, _KPW)
    da3 = _pad_i32(chain_da_3, _N).reshape(_NW, 2, _KPW // 2)
    s1 = _pad_f32(sign_1, _N).reshape(_NW, _KPW)
    s2 = _pad_f32(sign_2, _N).reshape(_NW, _KPW)
    s3 = _pad_f32(sign_3, _N).reshape(_NW, _KPW)
    out = _sc_loss(yp2, yt2, ba1, ba2, da1, da2, da3, s1, s2, s3)
    return jnp.sum(out)


# speedup vs baseline: 8.6870x; 1.1747x over previous
import functools

import jax
import jax.numpy as jnp
from jax import lax
from jax.experimental import pallas as pl
from jax.experimental.pallas import tpu as pltpu
from jax.experimental.pallas import tpu_sc as plsc

_B = 64
_N = 4096
_N_BA = 4095
_N_DA = 4094
_NW = 32
_KPW = _N // _NW
_L = 16
_NG = _B // _L
_D = 6 * _B


def _rsqrt(x):
    i = plsc.bitcast(x, jnp.int32)
    y = plsc.bitcast(jnp.int32(0x5F3759DF) - (i >> 1), jnp.float32)
    y = y * (1.5 - 0.5 * x * y * y)
    y = y * (1.5 - 0.5 * x * y * y)
    return y


def _cos_ba(a, b):
    ax, ay, az = a
    bx, by, bz = b
    inner = -(ax * bx + ay * by + az * bz)
    den = (ax * ax + ay * ay + az * az) * (bx * bx + by * by + bz * bz)
    c = inner * _rsqrt(den)
    return jnp.where(den == 0.0, jnp.zeros_like(c), c)


def _cos_da(a, b, c):
    ax, ay, az = a
    bx, by, bz = b
    cx, cy, cz = c
    c1x = ay * bz - az * by
    c1y = az * bx - ax * bz
    c1z = ax * by - ay * bx
    c2x = by * cz - bz * cy
    c2y = bz * cx - bx * cz
    c2z = bx * cy - by * cx
    inner = c1x * c2x + c1y * c2y + c1z * c2z
    den = (c1x * c1x + c1y * c1y + c1z * c1z) * (c2x * c2x + c2y * c2y + c2z * c2z)
    v = inner * _rsqrt(den)
    return jnp.where(den == 0.0, jnp.zeros_like(v), v)


def _p_vec(ref, k, g):
    o = g * _L
    return (ref[k, pl.ds(o, _L)],
            ref[k, pl.ds(_B + o, _L)],
            ref[k, pl.ds(2 * _B + o, _L)])


def _t_vec(ref, k, g):
    o = g * _L
    return (ref[k, pl.ds(3 * _B + o, _L)],
            ref[k, pl.ds(4 * _B + o, _L)],
            ref[k, pl.ds(5 * _B + o, _L)])


_mesh = plsc.VectorSubcoreMesh(core_axis_name="c", subcore_axis_name="s")


@functools.partial(
    pl.kernel,
    mesh=_mesh,
    compiler_params=pltpu.CompilerParams(needs_layout_passes=False,
                                         use_tc_tiling_on_sc=False),
    out_type=jax.ShapeDtypeStruct((_NW, _L), jnp.float32),
    scratch_types=[
        pltpu.VMEM((256, _D), jnp.float32),
        pltpu.VMEM((_KPW,), jnp.int32),
        pltpu.VMEM((_KPW,), jnp.int32),
        pltpu.VMEM((_KPW,), jnp.int32),
        pltpu.VMEM((_KPW,), jnp.int32),
        pltpu.VMEM((_KPW,), jnp.int32),
        pltpu.VMEM((_KPW,), jnp.float32),
        pltpu.VMEM((_KPW,), jnp.float32),
        pltpu.VMEM((_KPW,), jnp.float32),
        pltpu.VMEM((_KPW + _L,), jnp.float32),
        pltpu.VMEM((_L,), jnp.float32),
        pltpu.SemaphoreType.DMA,
        pltpu.SemaphoreType.DMA,
    ],
)
def _sc_loss(yc_hbm, ba1_hbm, ba2_hbm, da1_hbm, da2_hbm, da3_hbm,
             s1_hbm, s2_hbm, s3_hbm,
             out_hbm, BIG, I1, I2, J1, J2, J3, S1, S2, S3, FW, OB,
             sem, semi):
    cid = lax.axis_index("c")
    sid = lax.axis_index("s")
    wid = sid * 2 + cid
    base = wid * _KPW
    zero = jnp.zeros((_L,), jnp.float32)
    half = _KPW // 2
    quar = _KPW // 4

    pre = [pltpu.async_copy(ba1_hbm.at[wid], I1, semi),
           pltpu.async_copy(ba2_hbm.at[wid], I2, semi),
           pltpu.async_copy(da1_hbm.at[wid], J1, semi),
           pltpu.async_copy(da2_hbm.at[wid], J2, semi),
           pltpu.async_copy(da3_hbm.at[wid], J3, semi),
           pltpu.async_copy(s1_hbm.at[wid], S1, semi),
           pltpu.async_copy(s2_hbm.at[wid], S2, semi),
           pltpu.async_copy(s3_hbm.at[wid], S3, semi)]

    def issue_bl(h, r0):
        return [pltpu.async_copy(yc_hbm.at[pl.ds(base + h * half, half)],
                                 BIG.at[pl.ds(r0, half)], sem)]

    def issue_ba(h, r0, r1):
        return [
            pltpu.async_copy(yc_hbm.at[I1.at[pl.ds(h * half, half)]],
                             BIG.at[pl.ds(r0, half)], sem),
            pltpu.async_copy(yc_hbm.at[I2.at[pl.ds(h * half, half)]],
                             BIG.at[pl.ds(r1, half)], sem),
        ]

    def issue_da(q, r0, r1, r2):
        return [
            pltpu.async_copy(yc_hbm.at[J1.at[pl.ds(q * quar, quar)]],
                             BIG.at[pl.ds(r0, quar)], sem),
            pltpu.async_copy(yc_hbm.at[J2.at[pl.ds(q * quar, quar)]],
                             BIG.at[pl.ds(r1, quar)], sem),
            pltpu.async_copy(yc_hbm.at[J3.at[pl.ds(q * quar, quar)]],
                             BIG.at[pl.ds(r2, quar)], sem),
        ]

    def bl_compute(r0, acc):
        def body(k, acc):
            for g in range(_NG):
                px, py, pz = _p_vec(BIG, r0 + k, g)
                tx, ty, tz = _t_vec(BIG, r0 + k, g)
                sp = px * px + py * py + pz * pz
                st = tx * tx + ty * ty + tz * tz
                acc = acc + jnp.abs(st * _rsqrt(st) - sp * _rsqrt(sp))
            return acc
        return lax.fori_loop(0, half, body, acc)

    def ba_compute(h, r0, r1, acc):
        def body(k, acc):
            kacc = zero
            for g in range(_NG):
                kacc = kacc + jnp.abs(
                    _cos_ba(_t_vec(BIG, r0 + k, g), _t_vec(BIG, r1 + k, g))
                    - _cos_ba(_p_vec(BIG, r0 + k, g), _p_vec(BIG, r1 + k, g)))
            w = jnp.where(base + h * half + k < _N_BA, 1.0, 0.0)
            return acc + kacc * w.astype(jnp.float32)
        return lax.fori_loop(0, half, body, acc)

    def da_compute(q, r0, r1, r2, acc):
        def body(k, acc):
            kacc = zero
            for g in range(_NG):
                kacc = kacc + jnp.abs(
                    _cos_da(_t_vec(BIG, r0 + k, g), _t_vec(BIG, r1 + k, g),
                            _t_vec(BIG, r2 + k, g))
                    - _cos_da(_p_vec(BIG, r0 + k, g), _p_vec(BIG, r1 + k, g),
                              _p_vec(BIG, r2 + k, g)))
            kk = q * quar + k
            fw = FW[pl.ds(kk, _L)][0]
            w = jnp.where(base + kk < _N_DA, fw, 0.0)
            return acc + kacc * w.astype(jnp.float32)
        return lax.fori_loop(0, quar, body, acc)

    d_bl0 = issue_bl(0, 0)
    for cp in d_bl0:
        cp.wait()
    d_bl1 = issue_bl(1, 64)
    acc_bl = bl_compute(0, zero)
    for cp in d_bl1:
        cp.wait()
    for cp in pre:
        cp.wait()
    for c in range(_KPW // _L):
        o = c * _L
        sa = S1[pl.ds(o, _L)]
        sb = S2[pl.ds(o, _L)]
        sc = S3[pl.ds(o, _L)]
        num = jnp.abs(sa * sb * sb * sc)
        den = jnp.abs(sa * sb) * jnp.abs(sb * sc)
        safe = jnp.where(den == 0.0, jnp.ones_like(den), den)
        FW[pl.ds(o, _L)] = jnp.where(den == 0.0, jnp.zeros_like(num),
                                     num / safe)
    FW[pl.ds(_KPW, _L)] = zero

    d_ba0 = issue_ba(0, 128, 192)
    acc_bl = bl_compute(64, acc_bl)
    for cp in d_ba0:
        cp.wait()
    d_ba1 = issue_ba(1, 0, 64)
    acc_ba = ba_compute(0, 128, 192, zero)
    for cp in d_ba1:
        cp.wait()
    d_da0 = issue_da(0, 128, 160, 192)
    acc_ba = ba_compute(1, 0, 64, acc_ba)
    for cp in d_da0:
        cp.wait()
    d_da1 = issue_da(1, 0, 32, 64)
    acc_da = da_compute(0, 128, 160, 192, zero)
    for cp in d_da1:
        cp.wait()
    d_da2 = issue_da(2, 128, 160, 192)
    acc_da = da_compute(1, 0, 32, 64, acc_da)
    for cp in d_da2:
        cp.wait()
    d_da3 = issue_da(3, 0, 32, 64)
    acc_da = da_compute(2, 128, 160, 192, acc_da)
    for cp in d_da3:
        cp.wait()
    acc_da = da_compute(3, 0, 32, 64, acc_da)

    partial = (acc_bl * (1.0 / (_B * _N))
               + acc_ba * (1.0 / (_B * _N_BA))
               + acc_da * (1.0 / (_B * _N_DA)))
    OB[...] = partial
    pltpu.sync_copy(OB, out_hbm.at[wid])


def _pad_i32(a, n):
    return jnp.concatenate([a.astype(jnp.int32),
                            jnp.zeros((n - a.shape[0],), jnp.int32)])


def _pad_f32(a, n):
    return jnp.concatenate([a.astype(jnp.float32),
                            jnp.ones((n - a.shape[0],), jnp.float32)])


def kernel(y_p, y_t, chain_ba_1, chain_ba_2, chain_da_1, chain_da_2,
           chain_da_3, sign_1, sign_2, sign_3):
    yc = jnp.concatenate([y_p.transpose(1, 2, 0),
                          y_t.transpose(1, 2, 0)], axis=1).reshape(_N, _D)
    ba1 = _pad_i32(chain_ba_1, _N).reshape(_NW, _KPW)
    ba2 = _pad_i32(chain_ba_2, _N).reshape(_NW, _KPW)
    da1 = _pad_i32(chain_da_1, _N).reshape(_NW, _KPW)
    da2 = _pad_i32(chain_da_2, _N).reshape(_NW, _KPW)
    da3 = _pad_i32(chain_da_3, _N).reshape(_NW, _KPW)
    s1 = _pad_f32(sign_1, _N).reshape(_NW, _KPW)
    s2 = _pad_f32(sign_2, _N).reshape(_NW, _KPW)
    s3 = _pad_f32(sign_3, _N).reshape(_NW, _KPW)
    out = _sc_loss(yc, ba1, ba2, da1, da2, da3, s1, s2, s3)
    return jnp.sum(out)
